# baseline (speedup 1.0000x reference)
import jax
import jax.numpy as jnp
from jax.experimental import pallas as pl

N = 10000
E = 160000
OUT_DIM = 64
HEADS = 8
C = OUT_DIM * HEADS


def _identity_body(x_ref, o_ref):
    o_ref[...] = x_ref[...]


def kernel(q, k, v, edge_feat, edge_index, Wq, Wk, Wv, We, be, Wo, Wres,
           W1, bf1, W2, bf2, gamma1, beta1, gamma2, beta2):
    n = q.shape[0]
    src = edge_index[0]
    dst = edge_index[1]
    sqrt_dim = float(OUT_DIM) ** 0.5

    q_dst = (q @ Wq).reshape(n, HEADS, OUT_DIM)
    k_src = (k @ Wk).reshape(n, HEADS, OUT_DIM)
    v_src = (v @ Wv).reshape(n, HEADS, OUT_DIM)

    attn_score = jnp.sum(q_dst[dst] * k_src[src], axis=-1) / sqrt_dim
    edge_attn = edge_feat @ We + be
    attn_score = attn_score + edge_attn

    seg_max = jax.ops.segment_max(attn_score, dst, num_segments=n)
    seg_max = jnp.where(jnp.isfinite(seg_max), seg_max, 0.0)
    ex = jnp.exp(attn_score - jax.lax.stop_gradient(seg_max)[dst])
    denom = jax.ops.segment_sum(ex, dst, num_segments=n)
    attn = ex / denom[dst]

    msg = v_src[src] * attn[:, :, None]
    rst = jax.ops.segment_sum(msg, dst, num_segments=n)
    rst = rst.reshape(n, C)
    rst = rst @ Wo
    rst = rst + q @ Wres

    mean1 = jnp.mean(rst, axis=0)
    var1 = jnp.var(rst, axis=0)
    rst = (rst - mean1) / jnp.sqrt(var1 + 1e-5) * gamma1 + beta1

    q_in2 = rst
    h = jnp.maximum(rst @ W1 + bf1, 0.0)
    rst = h @ W2 + bf2
    rst = rst + q_in2

    mean2 = jnp.mean(rst, axis=0)
    var2 = jnp.var(rst, axis=0)
    rst = (rst - mean2) / jnp.sqrt(var2 + 1e-5) * gamma2 + beta2

    rst = pl.pallas_call(
        _identity_body,
        out_shape=jax.ShapeDtypeStruct(rst.shape, rst.dtype),
    )(rst)
    return rst


# speedup vs baseline: 9.5253x; 9.5253x over previous
import functools

import jax
import jax.numpy as jnp
import numpy as np
from jax import lax
from jax.experimental import pallas as pl
from jax.experimental.pallas import tpu as pltpu
from jax.experimental.pallas import tpu_sc as plsc

N = 10000
E = 160000
IN_DIM = 256
OUT_DIM = 64
HEADS = 8
C = OUT_DIM * HEADS

CH = 128
NCH = E // CH
NC, NS = 2, 16
NP = 10240
ROWS_PER_TILE = NP // NS

_MESH = plsc.VectorSubcoreMesh(core_axis_name="c", subcore_axis_name="s")

_NBLK = 400
_EBLK = 2000


def _proj_body(q_ref, k_ref, v_ref, wq_ref, wk_ref, wv_ref, wr_ref,
               Q_ref, K_ref, V_ref, R_ref):
    Q_ref[...] = jnp.dot(q_ref[...], wq_ref[...],
                         preferred_element_type=jnp.float32)
    K_ref[...] = jnp.dot(k_ref[...], wk_ref[...],
                         preferred_element_type=jnp.float32)
    V_ref[...] = jnp.dot(v_ref[...], wv_ref[...],
                         preferred_element_type=jnp.float32)
    R_ref[...] = jnp.dot(q_ref[...], wr_ref[...],
                         preferred_element_type=jnp.float32)


def _proj(q, k, v, Wq, Wk, Wv, Wres):
    nb = N // _NBLK
    blk = lambda i: (i, 0)
    w_spec = pl.BlockSpec((IN_DIM, C), lambda i: (0, 0))
    return pl.pallas_call(
        _proj_body,
        grid=(nb,),
        in_specs=[pl.BlockSpec((_NBLK, IN_DIM), blk)] * 3 + [w_spec] * 4,
        out_specs=[pl.BlockSpec((_NBLK, C), blk)] * 4,
        out_shape=[jax.ShapeDtypeStruct((N, C), jnp.float32)] * 4,
    )(q, k, v, Wq, Wk, Wv, Wres)


def _scores_body(qg_ref, kg_ref, ef_ref, wep_ref, bep_ref, m_ref, ex_ref):
    prod = qg_ref[...] * kg_ref[...]
    s = jnp.dot(prod, m_ref[...], preferred_element_type=jnp.float32)
    s = s * (1.0 / float(OUT_DIM) ** 0.5)
    s = s + jnp.dot(ef_ref[...], wep_ref[...],
                    preferred_element_type=jnp.float32) + bep_ref[...]
    ex_ref[...] = jnp.exp(s)


def _scores(Qg, Kg, ef, WeP, beP, M):
    nb = E // _EBLK
    blk = lambda i: (i, 0)
    return pl.pallas_call(
        _scores_body,
        grid=(nb,),
        in_specs=[pl.BlockSpec((_EBLK, C), blk),
                  pl.BlockSpec((_EBLK, C), blk),
                  pl.BlockSpec((_EBLK, 16), blk),
                  pl.BlockSpec((16, 128), lambda i: (0, 0)),
                  pl.BlockSpec((1, 128), lambda i: (0, 0)),
                  pl.BlockSpec((C, 128), lambda i: (0, 0))],
        out_specs=pl.BlockSpec((_EBLK, 128), blk),
        out_shape=jax.ShapeDtypeStruct((E, 128), jnp.float32),
    )(Qg, Kg, ef, WeP, beP, M)


def _rden_body(dp_ref, out_ref):
    den = dp_ref[0] + dp_ref[1]
    out_ref[...] = 1.0 / jnp.maximum(den, 1e-30)


def _rden(dpart):
    return pl.pallas_call(
        _rden_body,
        out_shape=jax.ShapeDtypeStruct((NP, 128), jnp.float32),
    )(dpart)


def _attnx_body(ex_ref, rd_ref, ax_ref):
    a = ex_ref[...] * rd_ref[...]
    parts = [jnp.broadcast_to(a[:, h:h + 1], (_EBLK, OUT_DIM))
             for h in range(HEADS)]
    ax_ref[...] = jnp.concatenate(parts, axis=1)


def _attnx(exP, rdenG):
    nb = E // _EBLK
    blk = lambda i: (i, 0)
    return pl.pallas_call(
        _attnx_body,
        grid=(nb,),
        in_specs=[pl.BlockSpec((_EBLK, 128), blk),
                  pl.BlockSpec((_EBLK, 128), blk)],
        out_specs=pl.BlockSpec((_EBLK, C), blk),
        out_shape=jax.ShapeDtypeStruct((E, C), jnp.float32),
    )(exP, rdenG)


def _head1_body(rst_ref, wo_ref, qres_ref, z_ref, s_ref, q_ref):
    z = jnp.dot(rst_ref[...], wo_ref[...],
                preferred_element_type=jnp.float32) + qres_ref[...]
    z_ref[...] = z

    @pl.when(pl.program_id(0) == 0)
    def _():
        s_ref[...] = jnp.zeros_like(s_ref)
        q_ref[...] = jnp.zeros_like(q_ref)

    s_ref[...] += jnp.sum(z, axis=0, keepdims=True)
    q_ref[...] += jnp.sum(z * z, axis=0, keepdims=True)


def _head1(rst, Wo, Qres):
    nb = N // _NBLK
    blk = lambda i: (i, 0)
    acc = pl.BlockSpec((1, C), lambda i: (0, 0))
    return pl.pallas_call(
        _head1_body,
        grid=(nb,),
        in_specs=[pl.BlockSpec((_NBLK, C), blk),
                  pl.BlockSpec((C, C), lambda i: (0, 0)),
                  pl.BlockSpec((_NBLK, C), blk)],
        out_specs=[pl.BlockSpec((_NBLK, C), blk), acc, acc],
        out_shape=[jax.ShapeDtypeStruct((N, C), jnp.float32),
                   jax.ShapeDtypeStruct((1, C), jnp.float32),
                   jax.ShapeDtypeStruct((1, C), jnp.float32)],
    )(rst, Wo, Qres)


def _head2_body(z_ref, a1_ref, b1_ref, w1_ref, bf1_ref, w2_ref, bf2_ref,
                y_ref, s_ref, q_ref):
    zn = z_ref[...] * a1_ref[...] + b1_ref[...]
    h = jnp.maximum(jnp.dot(zn, w1_ref[...],
                            preferred_element_type=jnp.float32)
                    + bf1_ref[...], 0.0)
    y = jnp.dot(h, w2_ref[...],
                preferred_element_type=jnp.float32) + bf2_ref[...] + zn
    y_ref[...] = y

    @pl.when(pl.program_id(0) == 0)
    def _():
        s_ref[...] = jnp.zeros_like(s_ref)
        q_ref[...] = jnp.zeros_like(q_ref)

    s_ref[...] += jnp.sum(y, axis=0, keepdims=True)
    q_ref[...] += jnp.sum(y * y, axis=0, keepdims=True)


def _head2(Z, a1, b1, W1, bf1, W2, bf2):
    nb = N // _NBLK
    blk = lambda i: (i, 0)
    acc = pl.BlockSpec((1, C), lambda i: (0, 0))
    one = lambda shape: pl.BlockSpec(shape, lambda i: (0, 0))
    return pl.pallas_call(
        _head2_body,
        grid=(nb,),
        in_specs=[pl.BlockSpec((_NBLK, C), blk),
                  one((1, C)), one((1, C)),
                  one((C, 2 * C)), one((1, 2 * C)),
                  one((2 * C, C)), one((1, C))],
        out_specs=[pl.BlockSpec((_NBLK, C), blk), acc, acc],
        out_shape=[jax.ShapeDtypeStruct((N, C), jnp.float32),
                   jax.ShapeDtypeStruct((1, C), jnp.float32),
                   jax.ShapeDtypeStruct((1, C), jnp.float32)],
    )(Z, a1, b1, W1, bf1, W2, bf2)


def _bn2_body(y_ref, a2_ref, b2_ref, out_ref):
    out_ref[...] = y_ref[...] * a2_ref[...] + b2_ref[...]


def _bn2(Y, a2, b2):
    nb = N // _NBLK
    blk = lambda i: (i, 0)
    return pl.pallas_call(
        _bn2_body,
        grid=(nb,),
        in_specs=[pl.BlockSpec((_NBLK, C), blk),
                  pl.BlockSpec((1, C), lambda i: (0, 0)),
                  pl.BlockSpec((1, C), lambda i: (0, 0))],
        out_specs=pl.BlockSpec((_NBLK, C), blk),
        out_shape=jax.ShapeDtypeStruct((N, C), jnp.float32),
    )(Y, a2, b2)


def _sc_gather_body(Q_hbm, K_hbm, dstR_hbm, srcR_hbm, Qg_hbm, Kg_hbm,
                    ibuf, buf):
    c = lax.axis_index("c")
    s = lax.axis_index("s")
    w = s * NC + c
    lo = w * 39 + jnp.minimum(w, 2)
    cnt = 39 + jnp.where(w < 2, 1, 0)

    @pl.loop(0, cnt)
    def _(i):
        r = lo + i
        base = r * CH
        pltpu.sync_copy(dstR_hbm.at[r], ibuf)
        pltpu.sync_copy(Q_hbm.at[ibuf.at[0]], buf)
        pltpu.sync_copy(buf, Qg_hbm.at[pl.ds(base, CH)])
        pltpu.sync_copy(srcR_hbm.at[r], ibuf)
        pltpu.sync_copy(K_hbm.at[ibuf.at[0]], buf)
        pltpu.sync_copy(buf, Kg_hbm.at[pl.ds(base, CH)])


def _sc_gather(Q, K, dstR, srcR):
    f = pl.kernel(
        _sc_gather_body,
        out_type=(jax.ShapeDtypeStruct((E, C), jnp.float32),
                  jax.ShapeDtypeStruct((E, C), jnp.float32)),
        mesh=_MESH,
        scratch_types=[pltpu.VMEM((1, CH), jnp.int32),
                       pltpu.VMEM((CH, C), jnp.float32)],
    )
    return f(Q, K, dstR, srcR)


def _sc_denom_body(exP_hbm, dstR_hbm, z128_hbm, dpart_hbm, ibuf, exbuf, acc):
    c = lax.axis_index("c")
    s = lax.axis_index("s")
    row0 = s * ROWS_PER_TILE
    pltpu.sync_copy(z128_hbm, acc.at[pl.ds(row0, ROWS_PER_TILE)])
    plsc.subcore_barrier()

    lo = c * (NCH // NC) + s * 39 + jnp.minimum(s, 1)
    cnt = 39 + jnp.where(s < 1, 1, 0)

    @pl.loop(0, cnt)
    def _(i):
        r = lo + i
        pltpu.sync_copy(dstR_hbm.at[r], ibuf)
        pltpu.sync_copy(exP_hbm.at[pl.ds(r * CH, CH)], exbuf)
        pltpu.sync_copy(exbuf, acc.at[ibuf.at[0]], add=True)

    plsc.subcore_barrier()
    pltpu.sync_copy(acc.at[pl.ds(row0, ROWS_PER_TILE)],
                    dpart_hbm.at[c, pl.ds(row0, ROWS_PER_TILE)])


def _sc_denom(exP, dstR, z128):
    f = pl.kernel(
        _sc_denom_body,
        out_type=jax.ShapeDtypeStruct((NC, NP, 128), jnp.float32),
        mesh=_MESH,
        scratch_types=[pltpu.VMEM((1, CH), jnp.int32),
                       pltpu.VMEM((CH, 128), jnp.float32),
                       pltpu.VMEM_SHARED((NP, 128), jnp.float32)],
    )
    return f(exP, dstR, z128)


def _sc_rdeng_body(rden_hbm, dstR_hbm, rdenG_hbm, ibuf, rbuf):
    c = lax.axis_index("c")
    s = lax.axis_index("s")
    w = s * NC + c
    lo = w * 39 + jnp.minimum(w, 2)
    cnt = 39 + jnp.where(w < 2, 1, 0)

    @pl.loop(0, cnt)
    def _(i):
        r = lo + i
        pltpu.sync_copy(dstR_hbm.at[r], ibuf)
        pltpu.sync_copy(rden_hbm.at[ibuf.at[0]], rbuf)
        pltpu.sync_copy(rbuf, rdenG_hbm.at[pl.ds(r * CH, CH)])


def _sc_rdeng(rden, dstR):
    f = pl.kernel(
        _sc_rdeng_body,
        out_type=jax.ShapeDtypeStruct((E, 128), jnp.float32),
        mesh=_MESH,
        scratch_types=[pltpu.VMEM((1, CH), jnp.int32),
                       pltpu.VMEM((CH, 128), jnp.float32)],
    )
    return f(rden, dstR)


def _sc_msg_body(VR_hbm, attnX_hbm, dstR_hbm, srcR_hbm, z128_hbm,
                 rst_hbm, dbuf, sbuf, i2buf, axbuf, vbuf, acc):
    c = lax.axis_index("c")
    s = lax.axis_index("s")
    row0 = s * ROWS_PER_TILE
    lo = s * 78 + jnp.minimum(s, 2)
    cnt = 78 + jnp.where(s < 2, 1, 0)

    for p in range(2):
        g = c * 2 + p
        pltpu.sync_copy(z128_hbm, acc.at[pl.ds(row0, ROWS_PER_TILE)])
        plsc.subcore_barrier()

        @pl.loop(0, cnt)
        def _(i):
            r = lo + i
            base = r * CH
            pltpu.sync_copy(dstR_hbm.at[r], dbuf)
            pltpu.sync_copy(srcR_hbm.at[r], sbuf)
            pltpu.sync_copy(attnX_hbm.at[pl.ds(base, CH),
                                         pl.ds(g * 128, 128)], axbuf)
            for j in range(CH // 16):
                sv = sbuf[0, pl.ds(j * 16, 16)]
                i2buf[0, pl.ds(j * 16, 16)] = sv * 4 + g
            pltpu.sync_copy(VR_hbm.at[i2buf.at[0]], vbuf)

            @pl.loop(0, CH)
            def _(j):
                for t in range(8):
                    sl = pl.ds(t * 16, 16)
                    vbuf[j, sl] = vbuf[j, sl] * axbuf[j, sl]

            pltpu.sync_copy(vbuf, acc.at[dbuf.at[0]], add=True)

        plsc.subcore_barrier()
        pltpu.sync_copy(acc.at[pl.ds(row0, ROWS_PER_TILE)],
                        rst_hbm.at[pl.ds(row0, ROWS_PER_TILE),
                                   pl.ds(g * 128, 128)])
        plsc.subcore_barrier()


def _sc_msg(VR, attnX, dstR, srcR, z128):
    f = pl.kernel(
        _sc_msg_body,
        out_type=jax.ShapeDtypeStruct((NP, C), jnp.float32),
        mesh=_MESH,
        scratch_types=[pltpu.VMEM((1, CH), jnp.int32),
                       pltpu.VMEM((1, CH), jnp.int32),
                       pltpu.VMEM((1, CH), jnp.int32),
                       pltpu.VMEM((CH, 128), jnp.float32),
                       pltpu.VMEM((CH, 128), jnp.float32),
                       pltpu.VMEM_SHARED((NP, 128), jnp.float32)],
    )
    return f(VR, attnX, dstR, srcR, z128)


def kernel(q, k, v, edge_feat, edge_index, Wq, Wk, Wv, We, be, Wo, Wres,
           W1, bf1, W2, bf2, gamma1, beta1, gamma2, beta2):
    src = edge_index[0].astype(jnp.int32)
    dst = edge_index[1].astype(jnp.int32)
    dstR = dst.reshape(NCH, 1, CH)
    srcR = src.reshape(NCH, 1, CH)

    WeP = jnp.concatenate([We, jnp.zeros((16, 120), jnp.float32)], axis=1)
    beP = jnp.concatenate([be, jnp.full((120,), -1e30, jnp.float32)])
    beP = beP.reshape(1, 128)

    m_np = np.zeros((C, 128), np.float32)
    for h in range(HEADS):
        m_np[h * OUT_DIM:(h + 1) * OUT_DIM, h] = 1.0
    M = jnp.asarray(m_np)

    z128 = jnp.zeros((ROWS_PER_TILE, 128), jnp.float32)

    Q, K, V, Qres = _proj(q, k, v, Wq, Wk, Wv, Wres)
    VR = V.reshape(N * 4, 128)

    Qg, Kg = _sc_gather(Q, K, dstR, srcR)
    exP = _scores(Qg, Kg, edge_feat, WeP, beP, M)

    dpart = _sc_denom(exP, dstR, z128)
    rden = _rden(dpart)
    rdenG = _sc_rdeng(rden, dstR)
    attnX = _attnx(exP, rdenG)

    rst = _sc_msg(VR, attnX, dstR, srcR, z128)
    rst = rst[:N]

    Z, s1, q1 = _head1(rst, Wo, Qres)
    mean1 = s1 / N
    var1 = q1 / N - mean1 * mean1
    a1 = gamma1 / jnp.sqrt(var1 + 1e-5)
    b1 = beta1 - mean1 * a1

    Y, s2, q2 = _head2(Z, a1, b1, W1, bf1.reshape(1, -1), W2,
                       bf2.reshape(1, -1))
    mean2 = s2 / N
    var2 = q2 / N - mean2 * mean2
    a2 = gamma2 / jnp.sqrt(var2 + 1e-5)
    b2 = beta2 - mean2 * a2

    return _bn2(Y, a2, b2)
